# baseline (speedup 1.0000x reference)
import jax
import jax.numpy as jnp
from jax.experimental import pallas as pl
from jax.experimental.pallas import tpu as pltpu

_K = 512
_D = 32
_SL = 32
_LN = 128
_NC = 4
_KC = _K // _NC
_KU = 8


def _vq_kernel(z_ref, emb_ref, out_ref, minv_ref, mini_ref):
    c = pl.program_id(0)

    def body(i, carry):
        minv, mini = carry
        k0 = i * _KU
        accs = [jnp.zeros((_SL, _LN), jnp.float32) for _ in range(_KU)]
        for d in range(_D):
            zd = z_ref[d]
            for j in range(_KU):
                t = zd - emb_ref[k0 + j, d]
                accs[j] = accs[j] + t * t
        for j in range(_KU):
            upd = accs[j] < minv
            minv = jnp.where(upd, accs[j], minv)
            mini = jnp.where(upd, c * _KC + k0 + j, mini)
        return minv, mini

    minv = jnp.full((_SL, _LN), jnp.inf, jnp.float32)
    mini = jnp.zeros((_SL, _LN), jnp.int32)
    for i in range(_KC // _KU):
        minv, mini = body(i, (minv, mini))

    @pl.when(c == 0)
    def _init():
        minv_ref[...] = minv
        mini_ref[...] = mini

    @pl.when(c != 0)
    def _fold():
        upd = minv < minv_ref[...]
        mini_ref[...] = jnp.where(upd, mini, mini_ref[...])
        minv_ref[...] = jnp.where(upd, minv, minv_ref[...])

    @pl.when(c == _NC - 1)
    def _emit():
        out_ref[...] = mini_ref[...]


def kernel(z_e_x, emb):
    B, D, H, W = z_e_x.shape
    zt = jnp.transpose(z_e_x, (1, 0, 2, 3)).reshape(D, _SL, _LN)
    out = pl.pallas_call(
        _vq_kernel,
        grid=(_NC,),
        in_specs=[
            pl.BlockSpec((D, _SL, _LN), lambda c: (0, 0, 0)),
            pl.BlockSpec((_KC, D), lambda c: (c, 0), memory_space=pltpu.SMEM),
        ],
        out_specs=pl.BlockSpec((_SL, _LN), lambda c: (0, 0)),
        out_shape=jax.ShapeDtypeStruct((_SL, _LN), jnp.int32),
        scratch_shapes=[
            pltpu.VMEM((_SL, _LN), jnp.float32),
            pltpu.VMEM((_SL, _LN), jnp.int32),
        ],
        compiler_params=pltpu.CompilerParams(
            dimension_semantics=("arbitrary",),
        ),
    )(zt, emb)
    return out.reshape(B, H, W)


# speedup vs baseline: 1.0092x; 1.0092x over previous
import jax
import jax.numpy as jnp
from jax.experimental import pallas as pl
from jax.experimental.pallas import tpu as pltpu

_K = 512
_D = 32
_SL = 32
_LN = 128
_NC = 8
_KC = _K // _NC
_KU = 8


def _vq_kernel(z_ref, emb_ref, out_ref, minv_ref, mini_ref):
    c = pl.program_id(0)

    def body(i, carry):
        minv, mini = carry
        k0 = i * _KU
        accs = [jnp.zeros((_SL, _LN), jnp.float32) for _ in range(_KU)]
        for d in range(_D):
            zd = z_ref[d]
            for j in range(_KU):
                t = zd - emb_ref[k0 + j, d]
                accs[j] = accs[j] + t * t
        for j in range(_KU):
            upd = accs[j] < minv
            minv = jnp.where(upd, accs[j], minv)
            mini = jnp.where(upd, c * _KC + k0 + j, mini)
        return minv, mini

    minv = jnp.full((_SL, _LN), jnp.inf, jnp.float32)
    mini = jnp.zeros((_SL, _LN), jnp.int32)
    for i in range(_KC // _KU):
        minv, mini = body(i, (minv, mini))

    @pl.when(c == 0)
    def _init():
        minv_ref[...] = minv
        mini_ref[...] = mini

    @pl.when(c != 0)
    def _fold():
        upd = minv < minv_ref[...]
        mini_ref[...] = jnp.where(upd, mini, mini_ref[...])
        minv_ref[...] = jnp.where(upd, minv, minv_ref[...])

    @pl.when(c == _NC - 1)
    def _emit():
        out_ref[...] = mini_ref[...]


def kernel(z_e_x, emb):
    B, D, H, W = z_e_x.shape
    zt = jnp.transpose(z_e_x, (1, 0, 2, 3)).reshape(D, _SL, _LN)
    out = pl.pallas_call(
        _vq_kernel,
        grid=(_NC,),
        in_specs=[
            pl.BlockSpec((D, _SL, _LN), lambda c: (0, 0, 0)),
            pl.BlockSpec((_KC, D), lambda c: (c, 0), memory_space=pltpu.SMEM),
        ],
        out_specs=pl.BlockSpec((_SL, _LN), lambda c: (0, 0)),
        out_shape=jax.ShapeDtypeStruct((_SL, _LN), jnp.int32),
        scratch_shapes=[
            pltpu.VMEM((_SL, _LN), jnp.float32),
            pltpu.VMEM((_SL, _LN), jnp.int32),
        ],
        compiler_params=pltpu.CompilerParams(
            dimension_semantics=("arbitrary",),
        ),
    )(zt, emb)
    return out.reshape(B, H, W)
